# Initial kernel scaffold; baseline (speedup 1.0000x reference)
import jax
import jax.numpy as jnp
from jax.experimental import pallas as pl


def kernel(logits):
    raise NotImplementedError("write your pallas kernel here")


# speedup vs baseline: 1.2733x; 1.2733x over previous
import functools

import jax
import jax.numpy as jnp
from jax.experimental import pallas as pl

B, D, K = 64, 32, 2048
N = B * D * K
HALF = N // 2

_KS0 = 0
_KS1 = 42
_KS2 = 0 ^ 42 ^ 0x1BD11BDA
_ROT_A = (13, 15, 26, 6)
_ROT_B = (17, 29, 16, 24)


def _rotl(x, r):
    return (x << jnp.uint32(r)) | (x >> jnp.uint32(32 - r))


def _rounds(x0, x1, rots):
    for r in rots:
        x0 = x0 + x1
        x1 = _rotl(x1, r)
        x1 = x1 ^ x0
    return x0, x1


def _threefry2x32(c0, c1):
    ks0, ks1, ks2 = jnp.uint32(_KS0), jnp.uint32(_KS1), jnp.uint32(_KS2)
    x0 = c0 + ks0
    x1 = c1 + ks1
    x0, x1 = _rounds(x0, x1, _ROT_A)
    x0 = x0 + ks1
    x1 = x1 + ks2 + jnp.uint32(1)
    x0, x1 = _rounds(x0, x1, _ROT_B)
    x0 = x0 + ks2
    x1 = x1 + ks0 + jnp.uint32(2)
    x0, x1 = _rounds(x0, x1, _ROT_A)
    x0 = x0 + ks0
    x1 = x1 + ks1 + jnp.uint32(3)
    x0, x1 = _rounds(x0, x1, _ROT_B)
    x0 = x0 + ks1
    x1 = x1 + ks2 + jnp.uint32(4)
    x0, x1 = _rounds(x0, x1, _ROT_A)
    x0 = x0 + ks2
    x1 = x1 + ks0 + jnp.uint32(5)
    return x0, x1


def _mc_kernel(l_ref, samp_ref, neg_ref):
    b = pl.program_id(0)
    l = l_ref[0]

    row = jax.lax.broadcasted_iota(jnp.uint32, (D, K), 0)
    col = jax.lax.broadcasted_iota(jnp.uint32, (D, K), 1)
    i = jnp.uint32(b) * jnp.uint32(D * K) + row * jnp.uint32(K) + col

    o0, o1 = _threefry2x32(jnp.zeros_like(i), i)
    bits = o0 ^ o1

    fbits = (bits >> jnp.uint32(9)) | jnp.uint32(0x3F800000)
    f01 = jax.lax.bitcast_convert_type(fbits, jnp.float32) - jnp.float32(1.0)
    minval = jnp.float32(1e-10)
    u = jnp.maximum(minval, f01 * (jnp.float32(1.0) - minval) + minval)
    gumbel = -jnp.log(-jnp.log(u))

    idx = jnp.argmax(l + gumbel, axis=-1)

    m = jnp.max(l, axis=-1, keepdims=True)
    lse = jnp.log(jnp.sum(jnp.exp(l - m), axis=-1))
    icol = jax.lax.broadcasted_iota(jnp.int32, (D, K), 1)
    l_at = jnp.sum(jnp.where(icol == idx[:, None], l, jnp.float32(0.0)), axis=-1)
    logp = l_at - m[:, 0] - lse

    samp_ref[...] = idx.reshape(1, 1, D)
    neg_ref[...] = (-jnp.sum(logp)).reshape(1, 1, 1)


@jax.jit
def kernel(logits):
    samples, neg = pl.pallas_call(
        _mc_kernel,
        grid=(B,),
        in_specs=[pl.BlockSpec((1, D, K), lambda b: (b, 0, 0))],
        out_specs=[
            pl.BlockSpec((1, 1, D), lambda b: (b, 0, 0)),
            pl.BlockSpec((1, 1, 1), lambda b: (b, 0, 0)),
        ],
        out_shape=[
            jax.ShapeDtypeStruct((B, 1, D), jnp.int32),
            jax.ShapeDtypeStruct((B, 1, 1), jnp.float32),
        ],
    )(logits)
    return samples.reshape(B, D), neg.reshape(B)


# speedup vs baseline: 1.2736x; 1.0003x over previous
import functools

import jax
import jax.numpy as jnp
from jax.experimental import pallas as pl
from jax.experimental.pallas import tpu as pltpu

B, D, K = 64, 32, 2048
N = B * D * K
HALF = N // 2

_KS0 = 0
_KS1 = 42
_KS2 = 0 ^ 42 ^ 0x1BD11BDA
_ROT_A = (13, 15, 26, 6)
_ROT_B = (17, 29, 16, 24)


def _rotl(x, r):
    return (x << jnp.uint32(r)) | (x >> jnp.uint32(32 - r))


def _rounds(x0, x1, rots):
    for r in rots:
        x0 = x0 + x1
        x1 = _rotl(x1, r)
        x1 = x1 ^ x0
    return x0, x1


def _threefry2x32(c0, c1):
    ks0, ks1, ks2 = jnp.uint32(_KS0), jnp.uint32(_KS1), jnp.uint32(_KS2)
    x0 = c0 + ks0
    x1 = c1 + ks1
    x0, x1 = _rounds(x0, x1, _ROT_A)
    x0 = x0 + ks1
    x1 = x1 + ks2 + jnp.uint32(1)
    x0, x1 = _rounds(x0, x1, _ROT_B)
    x0 = x0 + ks2
    x1 = x1 + ks0 + jnp.uint32(2)
    x0, x1 = _rounds(x0, x1, _ROT_A)
    x0 = x0 + ks0
    x1 = x1 + ks1 + jnp.uint32(3)
    x0, x1 = _rounds(x0, x1, _ROT_B)
    x0 = x0 + ks1
    x1 = x1 + ks2 + jnp.uint32(4)
    x0, x1 = _rounds(x0, x1, _ROT_A)
    x0 = x0 + ks2
    x1 = x1 + ks0 + jnp.uint32(5)
    return x0, x1


def _mc_kernel(l_ref, samp_ref, neg_ref):
    b = pl.program_id(0)
    l = l_ref[0]

    row = jax.lax.broadcasted_iota(jnp.uint32, (D, K), 0)
    col = jax.lax.broadcasted_iota(jnp.uint32, (D, K), 1)
    i = jnp.uint32(b) * jnp.uint32(D * K) + row * jnp.uint32(K) + col

    o0, o1 = _threefry2x32(jnp.zeros_like(i), i)
    bits = o0 ^ o1

    fbits = (bits >> jnp.uint32(9)) | jnp.uint32(0x3F800000)
    f01 = jax.lax.bitcast_convert_type(fbits, jnp.float32) - jnp.float32(1.0)
    minval = jnp.float32(1e-10)
    u = jnp.maximum(minval, f01 * (jnp.float32(1.0) - minval) + minval)
    gumbel = -jnp.log(-jnp.log(u))

    idx = jnp.argmax(l + gumbel, axis=-1)

    m = jnp.max(l, axis=-1, keepdims=True)
    lse = jnp.log(jnp.sum(jnp.exp(l - m), axis=-1))
    icol = jax.lax.broadcasted_iota(jnp.int32, (D, K), 1)
    l_at = jnp.sum(jnp.where(icol == idx[:, None], l, jnp.float32(0.0)), axis=-1)
    logp = l_at - m[:, 0] - lse

    samp_ref[...] = idx.reshape(1, 1, D)
    neg_ref[...] = (-jnp.sum(logp)).reshape(1, 1, 1)


@jax.jit
def kernel(logits):
    samples, neg = pl.pallas_call(
        _mc_kernel,
        grid=(B,),
        in_specs=[pl.BlockSpec((1, D, K), lambda b: (b, 0, 0))],
        out_specs=[
            pl.BlockSpec((1, 1, D), lambda b: (b, 0, 0)),
            pl.BlockSpec((1, 1, 1), lambda b: (b, 0, 0)),
        ],
        out_shape=[
            jax.ShapeDtypeStruct((B, 1, D), jnp.int32),
            jax.ShapeDtypeStruct((B, 1, 1), jnp.float32),
        ],
        compiler_params=pltpu.CompilerParams(
            dimension_semantics=("parallel",),
        ),
    )(logits)
    return samples.reshape(B, D), neg.reshape(B)


# speedup vs baseline: 1.4914x; 1.1710x over previous
import functools

import jax
import jax.numpy as jnp
from jax.experimental import pallas as pl
from jax.experimental.pallas import tpu as pltpu

B, D, K = 64, 32, 2048
N = B * D * K
HALF = N // 2

_KS0 = 0
_KS1 = 42
_KS2 = 0 ^ 42 ^ 0x1BD11BDA
_ROT_A = (13, 15, 26, 6)
_ROT_B = (17, 29, 16, 24)


def _rotl(x, r):
    return (x << jnp.uint32(r)) | (x >> jnp.uint32(32 - r))


def _rounds(x0, x1, rots):
    for r in rots:
        x0 = x0 + x1
        x1 = _rotl(x1, r)
        x1 = x1 ^ x0
    return x0, x1


def _threefry2x32_hi0(c1):
    ks0, ks1, ks2 = jnp.uint32(_KS0), jnp.uint32(_KS1), jnp.uint32(_KS2)
    x1 = c1 + ks1
    x0 = x1
    x1 = _rotl(x1, _ROT_A[0])
    x1 = x1 ^ x0
    x0, x1 = _rounds(x0, x1, _ROT_A[1:])
    x0 = x0 + ks1
    x1 = x1 + ks2 + jnp.uint32(1)
    x0, x1 = _rounds(x0, x1, _ROT_B)
    x0 = x0 + ks2
    x1 = x1 + ks0 + jnp.uint32(2)
    x0, x1 = _rounds(x0, x1, _ROT_A)
    x0 = x0 + ks0
    x1 = x1 + ks1 + jnp.uint32(3)
    x0, x1 = _rounds(x0, x1, _ROT_B)
    x0 = x0 + ks1
    x1 = x1 + ks2 + jnp.uint32(4)
    x0, x1 = _rounds(x0, x1, _ROT_A)
    x0 = x0 + ks2
    x1 = x1 + ks0 + jnp.uint32(5)
    return x0, x1


RB = 2
R = RB * D


def _mc_kernel(l_ref, samp_ref, neg_ref):
    b = pl.program_id(0)
    l = l_ref[...].reshape(R, K)

    row = jax.lax.broadcasted_iota(jnp.uint32, (R, K), 0)
    col = jax.lax.broadcasted_iota(jnp.uint32, (R, K), 1)
    i = jnp.uint32(b) * jnp.uint32(R * K) + row * jnp.uint32(K) + col

    o0, o1 = _threefry2x32_hi0(i)
    bits = o0 ^ o1

    fbits = (bits >> jnp.uint32(9)) | jnp.uint32(0x3F800000)
    f01 = jax.lax.bitcast_convert_type(fbits, jnp.float32) - jnp.float32(1.0)
    minval = jnp.float32(1e-10)
    u = f01 * (jnp.float32(1.0) - minval) + minval
    gumbel = -jnp.log(-jnp.log(u))

    idx = jnp.argmax(l + gumbel, axis=-1)

    m = jnp.max(l, axis=-1, keepdims=True)
    lse = jnp.log(jnp.sum(jnp.exp(l - m), axis=-1))
    icol = jax.lax.broadcasted_iota(jnp.int32, (R, K), 1)
    l_at = jnp.sum(jnp.where(icol == idx[:, None], l, jnp.float32(0.0)), axis=-1)
    logp = l_at - m[:, 0] - lse

    samp_ref[...] = idx.reshape(RB, 1, D)
    neg_ref[...] = (-jnp.sum(logp.reshape(RB, D), axis=1)).reshape(RB, 1, 1)


@jax.jit
def kernel(logits):
    samples, neg = pl.pallas_call(
        _mc_kernel,
        grid=(B // RB,),
        in_specs=[pl.BlockSpec((RB, D, K), lambda b: (b, 0, 0))],
        out_specs=[
            pl.BlockSpec((RB, 1, D), lambda b: (b, 0, 0)),
            pl.BlockSpec((RB, 1, 1), lambda b: (b, 0, 0)),
        ],
        out_shape=[
            jax.ShapeDtypeStruct((B, 1, D), jnp.int32),
            jax.ShapeDtypeStruct((B, 1, 1), jnp.float32),
        ],
        compiler_params=pltpu.CompilerParams(
            dimension_semantics=("parallel",),
        ),
    )(logits)
    return samples.reshape(B, D), neg.reshape(B)


# speedup vs baseline: 1.5394x; 1.0322x over previous
import functools

import jax
import jax.numpy as jnp
from jax.experimental import pallas as pl
from jax.experimental.pallas import tpu as pltpu

B, D, K = 64, 32, 2048
N = B * D * K
HALF = N // 2

_KS0 = 0
_KS1 = 42
_KS2 = 0 ^ 42 ^ 0x1BD11BDA
_ROT_A = (13, 15, 26, 6)
_ROT_B = (17, 29, 16, 24)


def _rotl(x, r):
    return (x << jnp.uint32(r)) | (x >> jnp.uint32(32 - r))


def _rounds(x0, x1, rots):
    for r in rots:
        x0 = x0 + x1
        x1 = _rotl(x1, r)
        x1 = x1 ^ x0
    return x0, x1


def _threefry2x32_hi0(c1):
    ks0, ks1, ks2 = jnp.uint32(_KS0), jnp.uint32(_KS1), jnp.uint32(_KS2)
    x1 = c1 + ks1
    x0 = x1
    x1 = _rotl(x1, _ROT_A[0])
    x1 = x1 ^ x0
    x0, x1 = _rounds(x0, x1, _ROT_A[1:])
    x0 = x0 + ks1
    x1 = x1 + ks2 + jnp.uint32(1)
    x0, x1 = _rounds(x0, x1, _ROT_B)
    x0 = x0 + ks2
    x1 = x1 + ks0 + jnp.uint32(2)
    x0, x1 = _rounds(x0, x1, _ROT_A)
    x0 = x0 + ks0
    x1 = x1 + ks1 + jnp.uint32(3)
    x0, x1 = _rounds(x0, x1, _ROT_B)
    x0 = x0 + ks1
    x1 = x1 + ks2 + jnp.uint32(4)
    x0, x1 = _rounds(x0, x1, _ROT_A)
    x0 = x0 + ks2
    x1 = x1 + ks0 + jnp.uint32(5)
    return x0, x1


RB = 4
R = RB * D


def _mc_kernel(l_ref, samp_ref, neg_ref):
    b = pl.program_id(0)
    l = l_ref[...].reshape(R, K)

    row = jax.lax.broadcasted_iota(jnp.uint32, (R, K), 0)
    col = jax.lax.broadcasted_iota(jnp.uint32, (R, K), 1)
    i = jnp.uint32(b) * jnp.uint32(R * K) + row * jnp.uint32(K) + col

    o0, o1 = _threefry2x32_hi0(i)
    bits = o0 ^ o1

    fbits = (bits >> jnp.uint32(9)) | jnp.uint32(0x3F800000)
    f01 = jax.lax.bitcast_convert_type(fbits, jnp.float32) - jnp.float32(1.0)
    minval = jnp.float32(1e-10)
    u = f01 * (jnp.float32(1.0) - minval) + minval
    gumbel = -jnp.log(-jnp.log(u))

    idx = jnp.argmax(l + gumbel, axis=-1)

    m = jnp.max(l, axis=-1, keepdims=True)
    lse = jnp.log(jnp.sum(jnp.exp(l - m), axis=-1))
    icol = jax.lax.broadcasted_iota(jnp.int32, (R, K), 1)
    l_at = jnp.sum(jnp.where(icol == idx[:, None], l, jnp.float32(0.0)), axis=-1)
    logp = l_at - m[:, 0] - lse

    samp_ref[...] = idx.reshape(RB, 1, D)
    neg_ref[...] = (-jnp.sum(logp.reshape(RB, D), axis=1)).reshape(RB, 1, 1)


@jax.jit
def kernel(logits):
    samples, neg = pl.pallas_call(
        _mc_kernel,
        grid=(B // RB,),
        in_specs=[pl.BlockSpec((RB, D, K), lambda b: (b, 0, 0))],
        out_specs=[
            pl.BlockSpec((RB, 1, D), lambda b: (b, 0, 0)),
            pl.BlockSpec((RB, 1, 1), lambda b: (b, 0, 0)),
        ],
        out_shape=[
            jax.ShapeDtypeStruct((B, 1, D), jnp.int32),
            jax.ShapeDtypeStruct((B, 1, 1), jnp.float32),
        ],
        compiler_params=pltpu.CompilerParams(
            dimension_semantics=("parallel",),
        ),
    )(logits)
    return samples.reshape(B, D), neg.reshape(B)


# speedup vs baseline: 1.5504x; 1.0071x over previous
import functools

import jax
import jax.numpy as jnp
from jax.experimental import pallas as pl
from jax.experimental.pallas import tpu as pltpu

B, D, K = 64, 32, 2048
N = B * D * K
HALF = N // 2

_KS0 = 0
_KS1 = 42
_KS2 = 0 ^ 42 ^ 0x1BD11BDA
_ROT_A = (13, 15, 26, 6)
_ROT_B = (17, 29, 16, 24)


def _rotl(x, r):
    return (x << jnp.uint32(r)) | (x >> jnp.uint32(32 - r))


def _rounds(x0, x1, rots):
    for r in rots:
        x0 = x0 + x1
        x1 = _rotl(x1, r)
        x1 = x1 ^ x0
    return x0, x1


def _threefry2x32_hi0(c1):
    ks0, ks1, ks2 = jnp.uint32(_KS0), jnp.uint32(_KS1), jnp.uint32(_KS2)
    x1 = c1 + ks1
    x0 = x1
    x1 = _rotl(x1, _ROT_A[0])
    x1 = x1 ^ x0
    x0, x1 = _rounds(x0, x1, _ROT_A[1:])
    x0 = x0 + ks1
    x1 = x1 + ks2 + jnp.uint32(1)
    x0, x1 = _rounds(x0, x1, _ROT_B)
    x0 = x0 + ks2
    x1 = x1 + ks0 + jnp.uint32(2)
    x0, x1 = _rounds(x0, x1, _ROT_A)
    x0 = x0 + ks0
    x1 = x1 + ks1 + jnp.uint32(3)
    x0, x1 = _rounds(x0, x1, _ROT_B)
    x0 = x0 + ks1
    x1 = x1 + ks2 + jnp.uint32(4)
    x0, x1 = _rounds(x0, x1, _ROT_A)
    x0 = x0 + ks2
    x1 = x1 + ks0 + jnp.uint32(5)
    return x0, x1


RB = 8
R = RB * D


def _mc_kernel(l_ref, samp_ref, neg_ref):
    b = pl.program_id(0)
    l = l_ref[...].reshape(R, K)

    row = jax.lax.broadcasted_iota(jnp.uint32, (R, K), 0)
    col = jax.lax.broadcasted_iota(jnp.uint32, (R, K), 1)
    i = jnp.uint32(b) * jnp.uint32(R * K) + row * jnp.uint32(K) + col

    o0, o1 = _threefry2x32_hi0(i)
    bits = o0 ^ o1

    fbits = (bits >> jnp.uint32(9)) | jnp.uint32(0x3F800000)
    f01 = jax.lax.bitcast_convert_type(fbits, jnp.float32) - jnp.float32(1.0)
    minval = jnp.float32(1e-10)
    u = f01 * (jnp.float32(1.0) - minval) + minval
    gumbel = -jnp.log(-jnp.log(u))

    idx = jnp.argmax(l + gumbel, axis=-1)

    m = jnp.max(l, axis=-1, keepdims=True)
    lse = jnp.log(jnp.sum(jnp.exp(l - m), axis=-1))
    icol = jax.lax.broadcasted_iota(jnp.int32, (R, K), 1)
    l_at = jnp.sum(jnp.where(icol == idx[:, None], l, jnp.float32(0.0)), axis=-1)
    logp = l_at - m[:, 0] - lse

    samp_ref[...] = idx.reshape(RB, 1, D)
    neg_ref[...] = (-jnp.sum(logp.reshape(RB, D), axis=1)).reshape(RB, 1, 1)


@jax.jit
def kernel(logits):
    samples, neg = pl.pallas_call(
        _mc_kernel,
        grid=(B // RB,),
        in_specs=[pl.BlockSpec((RB, D, K), lambda b: (b, 0, 0))],
        out_specs=[
            pl.BlockSpec((RB, 1, D), lambda b: (b, 0, 0)),
            pl.BlockSpec((RB, 1, 1), lambda b: (b, 0, 0)),
        ],
        out_shape=[
            jax.ShapeDtypeStruct((B, 1, D), jnp.int32),
            jax.ShapeDtypeStruct((B, 1, 1), jnp.float32),
        ],
        compiler_params=pltpu.CompilerParams(
            dimension_semantics=("parallel",),
        ),
    )(logits)
    return samples.reshape(B, D), neg.reshape(B)
